# baseline (speedup 1.0000x reference)
import jax
import jax.numpy as jnp
from jax.experimental import pallas as pl
from jax.experimental.pallas import tpu as pltpu

B, C, H, W = 128, 3, 256, 256
ROWLEN = C * H * W
LANES = 128
ROWS = ROWLEN // LANES
NCH = 4
CHUNK = ROWS // NCH


def _qsample_kernel(ts_ref, g_ref, og_ref, x_ref, n_ref, o_ref):
    b = pl.program_id(0)
    t = ts_ref[b]
    a = g_ref[t]
    c = og_ref[t]
    o_ref[...] = a * x_ref[...] + c * n_ref[...]


def kernel(x_start, timesteps, noise, sqrt_gammas, sqrt_one_minus_gammas):
    ts = timesteps.astype(jnp.int32)
    g = sqrt_gammas.reshape(-1)
    og = sqrt_one_minus_gammas.reshape(-1)
    x3 = x_start.reshape(B, ROWS, LANES)
    n3 = noise.reshape(B, ROWS, LANES)

    out = pl.pallas_call(
        _qsample_kernel,
        grid=(B, NCH),
        in_specs=[
            pl.BlockSpec(memory_space=pltpu.SMEM),
            pl.BlockSpec(memory_space=pltpu.SMEM),
            pl.BlockSpec(memory_space=pltpu.SMEM),
            pl.BlockSpec((1, CHUNK, LANES), lambda b, j: (b, j, 0)),
            pl.BlockSpec((1, CHUNK, LANES), lambda b, j: (b, j, 0)),
        ],
        out_specs=pl.BlockSpec((1, CHUNK, LANES), lambda b, j: (b, j, 0)),
        out_shape=jax.ShapeDtypeStruct((B, ROWS, LANES), jnp.float32),
    )(ts, g, og, x3, n3)

    return out.reshape(B, C, H, W)


# speedup vs baseline: 1.5725x; 1.5725x over previous
import jax
import jax.numpy as jnp
from jax.experimental import pallas as pl
from jax.experimental.pallas import tpu as pltpu

B, C, H, W = 128, 3, 256, 256
ROWLEN = C * H * W
LANES = 128
ROWS = ROWLEN // LANES
BPB = 4


def _qsample_kernel(ts_ref, g_ref, og_ref, x_ref, n_ref, o_ref):
    blk = pl.program_id(0)
    for i in range(BPB):
        t = ts_ref[blk * BPB + i]
        a = g_ref[t]
        c = og_ref[t]
        o_ref[i] = a * x_ref[i] + c * n_ref[i]


def kernel(x_start, timesteps, noise, sqrt_gammas, sqrt_one_minus_gammas):
    ts = timesteps.astype(jnp.int32)
    g = sqrt_gammas.reshape(-1)
    og = sqrt_one_minus_gammas.reshape(-1)
    x3 = x_start.reshape(B, ROWS, LANES)
    n3 = noise.reshape(B, ROWS, LANES)

    out = pl.pallas_call(
        _qsample_kernel,
        grid=(B // BPB,),
        in_specs=[
            pl.BlockSpec(memory_space=pltpu.SMEM),
            pl.BlockSpec(memory_space=pltpu.SMEM),
            pl.BlockSpec(memory_space=pltpu.SMEM),
            pl.BlockSpec((BPB, ROWS, LANES), lambda b: (b, 0, 0)),
            pl.BlockSpec((BPB, ROWS, LANES), lambda b: (b, 0, 0)),
        ],
        out_specs=pl.BlockSpec((BPB, ROWS, LANES), lambda b: (b, 0, 0)),
        out_shape=jax.ShapeDtypeStruct((B, ROWS, LANES), jnp.float32),
        compiler_params=pltpu.CompilerParams(
            dimension_semantics=("arbitrary",),
        ),
    )(ts, g, og, x3, n3)

    return out.reshape(B, C, H, W)
